# baseline (speedup 1.0000x reference)
import functools

import jax
import jax.numpy as jnp
from jax import lax
from jax.experimental import pallas as pl
from jax.experimental.pallas import tpu as pltpu
from jax.experimental.pallas import tpu_sc as plsc

NC = 2
TPS = 16
CH = 256


def _make_sc_propagate(N, HD, K, N_pad, CPT):
    R = (K + 1) * NC * N_pad
    RPT = N_pad // TPS

    mesh = plsc.VectorSubcoreMesh(core_axis_name="c", subcore_axis_name="s")

    @functools.partial(
        pl.kernel,
        out_type=jax.ShapeDtypeStruct((R, HD), jnp.float32),
        mesh=mesh,
        scratch_types=[
            pltpu.VMEM((CPT, CH), jnp.int32),
            pltpu.VMEM((CPT, CH), jnp.int32),
            pltpu.VMEM((CH, HD), jnp.float32),
            pltpu.VMEM((CH, HD), jnp.float32),
            pltpu.VMEM_SHARED((N_pad, HD), jnp.float32),
            pltpu.SemaphoreType.DMA,
            pltpu.SemaphoreType.DMA,
        ],
        compiler_params=pltpu.CompilerParams(use_tc_tiling_on_sc=False),
    )
    def body(xs_hbm, srcb_hbm, dstb_hbm, zeros_hbm, hs_hbm,
             src_v, dst_v, rows0_v, rows1_v, accum, sem0, sem1):
        c = lax.axis_index("c")
        s = lax.axis_index("s")
        base_row = s * RPT

        pltpu.sync_copy(srcb_hbm.at[s], src_v)
        pltpu.sync_copy(dstb_hbm.at[s], dst_v)

        for q in range(RPT // CH):
            r0 = base_row + q * CH
            pltpu.sync_copy(xs_hbm.at[c, pl.ds(r0, CH)], rows0_v)
            pltpu.sync_copy(rows0_v, hs_hbm.at[pl.ds(c * N_pad + r0, CH)])
        pltpu.sync_copy(zeros_hbm, accum.at[pl.ds(base_row, RPT)])
        plsc.subcore_barrier()

        def hop(k, carry):
            boff = ((k - 1) * NC + c) * N_pad
            hcur = hs_hbm.at[pl.ds(boff, N_pad)]

            pltpu.async_copy(hcur.at[src_v.at[0]], rows0_v, sem0)

            def chunk2(i, cc):
                j0 = 2 * i
                j1 = j0 + 1
                pltpu.async_copy(hcur.at[src_v.at[j1]], rows1_v, sem1)
                pltpu.make_async_copy(hcur.at[src_v.at[j0]], rows0_v,
                                      sem0).wait()
                pltpu.sync_copy(rows0_v, accum.at[dst_v.at[j0]], add=True)

                @pl.when(j1 + 1 < CPT)
                def _():
                    pltpu.async_copy(hcur.at[src_v.at[j1 + 1]], rows0_v, sem0)

                pltpu.make_async_copy(hcur.at[src_v.at[j1]], rows1_v,
                                      sem1).wait()
                pltpu.sync_copy(rows1_v, accum.at[dst_v.at[j1]], add=True)
                return cc

            lax.fori_loop(0, CPT // 2, chunk2, 0)
            plsc.subcore_barrier()

            orow = (k * NC + c) * N_pad + base_row
            pltpu.sync_copy(accum.at[pl.ds(base_row, RPT)],
                            hs_hbm.at[pl.ds(orow, RPT)])
            pltpu.sync_copy(zeros_hbm, accum.at[pl.ds(base_row, RPT)])
            plsc.subcore_barrier()
            return carry

        lax.fori_loop(1, K + 1, hop, 0)

    return body


def _make_tc_combine(N, D, HD, K, N_pad):
    BN = 1000

    def body(att_ref, hs_ref, out_ref):
        a = att_ref[...]
        m = jnp.max(a, axis=-1, keepdims=True)
        e = jnp.exp(a - m)
        w = e / jnp.sum(e, axis=-1, keepdims=True)
        halves = []
        for cc in range(NC):
            acc = jnp.zeros((BN, HD), jnp.float32)
            for k in range(K + 1):
                wk = w[:, k:k + 1]
                acc = acc + wk * hs_ref[k, cc]
            halves.append(acc)
        out_ref[...] = jnp.concatenate(halves, axis=-1)

    return pl.pallas_call(
        body,
        grid=(N // BN,),
        in_specs=[
            pl.BlockSpec((1, K + 1), lambda i: (0, 0)),
            pl.BlockSpec((K + 1, NC, BN, HD), lambda i: (0, 0, i, 0)),
        ],
        out_specs=pl.BlockSpec((BN, D), lambda i: (i, 0)),
        out_shape=jax.ShapeDtypeStruct((N, D), jnp.float32),
    )


def kernel(x, edge_index, att):
    N, D = x.shape
    E = edge_index.shape[1]
    K = att.shape[0] - 1
    HD = D // NC
    N_pad = -(-(N + 1) // (TPS * CH)) * (TPS * CH)
    CPT = -(-E // (TPS * CH))
    CPT += CPT % 2
    E_pad = TPS * CPT * CH

    src = edge_index[0]
    dst = edge_index[1]
    pad = E_pad - E
    src_p = jnp.concatenate([src, jnp.zeros((pad,), jnp.int32)])
    dst_p = jnp.concatenate([dst, jnp.full((pad,), N, jnp.int32)])
    srcb = src_p.reshape(TPS, CPT, CH)
    dstb = dst_p.reshape(TPS, CPT, CH)

    xs = jnp.stack([x[:, :HD], x[:, HD:]])
    xs = jnp.pad(xs, ((0, 0), (0, N_pad - N), (0, 0)))
    zeros = jnp.zeros((N_pad // TPS, HD), jnp.float32)

    hs = _make_sc_propagate(N, HD, K, N_pad, CPT)(xs, srcb, dstb, zeros)
    hs4 = hs.reshape(K + 1, NC, N_pad, HD)
    out = _make_tc_combine(N, D, HD, K, N_pad)(att.reshape(1, K + 1), hs4)
    return out


# speedup vs baseline: 1.3780x; 1.3780x over previous
import functools

import jax
import jax.numpy as jnp
from jax import lax
from jax.experimental import pallas as pl
from jax.experimental.pallas import tpu as pltpu
from jax.experimental.pallas import tpu_sc as plsc

NC = 2
TPS = 16
CH = 128


def _make_sc_propagate(N, HD, K, N_pad, CPT):
    R = (K + 1) * NC * N_pad
    RPT = N_pad // TPS

    mesh = plsc.VectorSubcoreMesh(core_axis_name="c", subcore_axis_name="s")

    @functools.partial(
        pl.kernel,
        out_type=jax.ShapeDtypeStruct((R, HD), jnp.float32),
        mesh=mesh,
        scratch_types=[
            pltpu.VMEM((CPT, CH), jnp.int32),
            pltpu.VMEM((CPT, CH), jnp.int32),
            pltpu.VMEM((CH, HD), jnp.float32),
            pltpu.VMEM((CH, HD), jnp.float32),
            pltpu.VMEM_SHARED((N_pad, HD), jnp.float32),
            pltpu.SemaphoreType.DMA,
            pltpu.SemaphoreType.DMA,
        ],
        compiler_params=pltpu.CompilerParams(use_tc_tiling_on_sc=False),
    )
    def body(xs_hbm, srcb_hbm, dstb_hbm, zeros_hbm, hs_hbm,
             src_v, dst_v, rows0_v, rows1_v, accum, sem0, sem1):
        c = lax.axis_index("c")
        s = lax.axis_index("s")
        base_row = s * RPT

        pltpu.sync_copy(dstb_hbm.at[s], dst_v)

        for q in range(RPT // CH):
            r0 = base_row + q * CH
            pltpu.sync_copy(xs_hbm.at[c, pl.ds(r0, CH)], rows0_v)
            pltpu.sync_copy(rows0_v, hs_hbm.at[pl.ds(c * N_pad + r0, CH)])
        rem = RPT % CH
        if rem:
            r0 = base_row + (RPT // CH) * CH
            pltpu.sync_copy(xs_hbm.at[c, pl.ds(r0, rem)],
                            rows0_v.at[pl.ds(0, rem)])
            pltpu.sync_copy(rows0_v.at[pl.ds(0, rem)],
                            hs_hbm.at[pl.ds(c * N_pad + r0, rem)])
        pltpu.sync_copy(zeros_hbm, accum.at[pl.ds(base_row, RPT)])
        plsc.subcore_barrier()

        def hop(k, carry):
            pltpu.sync_copy(srcb_hbm.at[k - 1, c, s], src_v)

            pltpu.async_copy(hs_hbm.at[src_v.at[0]], rows0_v, sem0)

            def chunk2(i, cc):
                j0 = 2 * i
                j1 = j0 + 1
                pltpu.async_copy(hs_hbm.at[src_v.at[j1]], rows1_v, sem1)
                pltpu.make_async_copy(hs_hbm.at[src_v.at[j0]], rows0_v,
                                      sem0).wait()
                pltpu.sync_copy(rows0_v, accum.at[dst_v.at[j0]], add=True)

                @pl.when(j1 + 1 < CPT)
                def _():
                    pltpu.async_copy(hs_hbm.at[src_v.at[j1 + 1]], rows0_v, sem0)

                pltpu.make_async_copy(hs_hbm.at[src_v.at[j1]], rows1_v,
                                      sem1).wait()
                pltpu.sync_copy(rows1_v, accum.at[dst_v.at[j1]], add=True)
                return cc

            lax.fori_loop(0, CPT // 2, chunk2, 0)
            plsc.subcore_barrier()

            orow = (k * NC + c) * N_pad + base_row
            pltpu.sync_copy(accum.at[pl.ds(base_row, RPT)],
                            hs_hbm.at[pl.ds(orow, RPT)])
            pltpu.sync_copy(zeros_hbm, accum.at[pl.ds(base_row, RPT)])
            plsc.subcore_barrier()
            return carry

        lax.fori_loop(1, K + 1, hop, 0)

    return body


def _make_tc_combine(N, D, HD, K, N_pad):
    BN = 1000

    def body(att_ref, hs_ref, out_ref):
        a = att_ref[...]
        m = jnp.max(a, axis=-1, keepdims=True)
        e = jnp.exp(a - m)
        w = e / jnp.sum(e, axis=-1, keepdims=True)
        halves = []
        for cc in range(NC):
            acc = jnp.zeros((BN, HD), jnp.float32)
            for k in range(K + 1):
                wk = w[:, k:k + 1]
                acc = acc + wk * hs_ref[k, cc]
            halves.append(acc)
        out_ref[...] = jnp.concatenate(halves, axis=-1)

    return pl.pallas_call(
        body,
        grid=(N // BN,),
        in_specs=[
            pl.BlockSpec((1, K + 1), lambda i: (0, 0)),
            pl.BlockSpec((K + 1, NC, BN, HD), lambda i: (0, 0, i, 0)),
        ],
        out_specs=pl.BlockSpec((BN, D), lambda i: (i, 0)),
        out_shape=jax.ShapeDtypeStruct((N, D), jnp.float32),
    )


def kernel(x, edge_index, att):
    N, D = x.shape
    E = edge_index.shape[1]
    K = att.shape[0] - 1
    HD = D // NC
    N_pad = -(-(N + 1) // (TPS * 8)) * (TPS * 8)
    CPT = -(-E // (TPS * CH))
    CPT += CPT % 2
    E_pad = TPS * CPT * CH

    src = edge_index[0]
    dst = edge_index[1]
    pad = E_pad - E
    src_p = jnp.concatenate([src, jnp.zeros((pad,), jnp.int32)])
    dst_p = jnp.concatenate([dst, jnp.full((pad,), N, jnp.int32)])

    bias = (jnp.arange(K)[:, None] * NC + jnp.arange(NC)[None, :]) * N_pad
    srcb = (bias[:, :, None] + src_p[None, None, :]).reshape(K, NC, TPS, CPT, CH)
    dstb = dst_p.reshape(TPS, CPT, CH)

    xs = jnp.stack([x[:, :HD], x[:, HD:]])
    xs = jnp.pad(xs, ((0, 0), (0, N_pad - N), (0, 0)))
    zeros = jnp.zeros((N_pad // TPS, HD), jnp.float32)

    hs = _make_sc_propagate(N, HD, K, N_pad, CPT)(
        xs, srcb.astype(jnp.int32), dstb, zeros)
    hs4 = hs.reshape(K + 1, NC, N_pad, HD)
    out = _make_tc_combine(N, D, HD, K, N_pad)(att.reshape(1, K + 1), hs4)
    return out
